# baseline (speedup 1.0000x reference)
import functools

import jax
import jax.numpy as jnp
import numpy as np
from jax import lax
from jax.experimental import pallas as pl
from jax.experimental.pallas import tpu as pltpu
from jax.experimental.pallas import tpu_sc as plsc

D_IN = 768
E = 8
HIDDEN = 256
N_TOK = 4096
EPS = 0.1
_SQRT_HALF = 0.7071067811865476

BLK_S = 256
G = N_TOK * 2 // BLK_S + E
P = G * BLK_S
CHUNK = 512
NW = 32
TPW = N_TOK // NW
WREP = 128


def _router_kernel(x_ref, wr_ref, br_ref, pos0_ref, pos1_ref, w0_ref,
                   w1_ref, eob_ref, aux_ref):
    x = x_ref[...]
    logits = jnp.dot(x, wr_ref[...], preferred_element_type=jnp.float32)
    logits = logits + br_ref[...]
    m = jnp.max(logits, axis=-1, keepdims=True)
    ex = jnp.exp(logits - m)
    probs = ex / jnp.sum(ex, axis=-1, keepdims=True)
    probs = (1.0 - EPS) * probs + EPS / E

    iota = jax.lax.broadcasted_iota(jnp.int32, probs.shape, 1)
    m1 = jnp.max(probs, axis=-1, keepdims=True)
    e1 = jnp.min(jnp.where(probs == m1, iota, E), axis=-1, keepdims=True)
    probs2 = jnp.where(iota == e1, -jnp.inf, probs)
    m2 = jnp.max(probs2, axis=-1, keepdims=True)
    e2 = jnp.min(jnp.where(probs2 == m2, iota, E), axis=-1, keepdims=True)

    load = jnp.sum(probs, axis=0, keepdims=True) / N_TOK
    aux = jnp.sum(load * jnp.log(load * E + 1e-9)) / np.log(E + 1e-9)
    aux_ref[...] = jnp.reshape(aux, (1, 1))

    oh1 = (iota == e1).astype(jnp.float32)
    oh2 = (iota == e2).astype(jnp.float32)
    C = oh1 + oh2

    r = jax.lax.broadcasted_iota(jnp.int32, (CHUNK, CHUNK), 0)
    c = jax.lax.broadcasted_iota(jnp.int32, (CHUNK, CHUNK), 1)
    T = (r > c).astype(jnp.float32)
    run = jnp.zeros((1, E), jnp.float32)
    chunks = []
    for b in range(N_TOK // CHUNK):
        Cb = C[b * CHUNK:(b + 1) * CHUNK]
        chunks.append(jnp.dot(T, Cb, preferred_element_type=jnp.float32) + run)
        run = run + jnp.sum(Cb, axis=0, keepdims=True)
    Pm = jnp.concatenate(chunks, axis=0)

    padded = jnp.ceil(run / BLK_S) * BLK_S
    ri = jax.lax.broadcasted_iota(jnp.int32, (E, E), 0)
    ci = jax.lax.broadcasted_iota(jnp.int32, (E, E), 1)
    TU = (ri < ci).astype(jnp.float32)
    ps = jnp.dot(padded, TU, preferred_element_type=jnp.float32)

    base = ps + Pm
    pos0 = jnp.sum(oh1 * base, axis=1, keepdims=True)
    pos1 = jnp.sum(oh2 * base, axis=1, keepdims=True)
    pos0_ref[...] = pos0.astype(jnp.int32)
    pos1_ref[...] = pos1.astype(jnp.int32)
    w0_ref[...] = jnp.broadcast_to(m1, (N_TOK, WREP))
    w1_ref[...] = jnp.broadcast_to(m2, (N_TOK, WREP))

    pad_end = ps + padded
    gi = jax.lax.broadcasted_iota(
        jnp.int32, (1, 64), 1).astype(jnp.float32) * BLK_S
    acc = jnp.zeros((1, 64), jnp.float32)
    for e in range(E):
        acc = acc + (gi >= pad_end[0:1, e:e + 1]).astype(jnp.float32)
    eob_ref[...] = jnp.minimum(acc, E - 1).astype(jnp.int32)


def _router(x, Wr, br):
    return pl.pallas_call(
        _router_kernel,
        grid=(1,),
        in_specs=[
            pl.BlockSpec((N_TOK, D_IN), lambda g: (0, 0)),
            pl.BlockSpec((D_IN, E), lambda g: (0, 0)),
            pl.BlockSpec((E,), lambda g: (0,)),
        ],
        out_specs=[
            pl.BlockSpec((N_TOK, 1), lambda g: (0, 0)),
            pl.BlockSpec((N_TOK, 1), lambda g: (0, 0)),
            pl.BlockSpec((N_TOK, WREP), lambda g: (0, 0)),
            pl.BlockSpec((N_TOK, WREP), lambda g: (0, 0)),
            pl.BlockSpec((1, 64), lambda g: (0, 0)),
            pl.BlockSpec((1, 1), lambda g: (0, 0)),
        ],
        out_shape=[
            jax.ShapeDtypeStruct((N_TOK, 1), jnp.int32),
            jax.ShapeDtypeStruct((N_TOK, 1), jnp.int32),
            jax.ShapeDtypeStruct((N_TOK, WREP), jnp.float32),
            jax.ShapeDtypeStruct((N_TOK, WREP), jnp.float32),
            jax.ShapeDtypeStruct((1, 64), jnp.int32),
            jax.ShapeDtypeStruct((1, 1), jnp.float32),
        ],
    )(x, Wr, br)


_SC_MESH = plsc.VectorSubcoreMesh(core_axis_name="c", subcore_axis_name="s")


@functools.partial(
    pl.kernel,
    mesh=_SC_MESH,
    out_type=[
        jax.ShapeDtypeStruct((P, D_IN), jnp.float32),
        jax.ShapeDtypeStruct((P, WREP), jnp.float32),
    ],
    scratch_types=[
        pltpu.VMEM((2, TPW // 2), jnp.int32),
        pltpu.VMEM((2, TPW // 2), jnp.int32),
        pltpu.VMEM((TPW // 2, D_IN), jnp.float32),
        pltpu.VMEM((TPW // 2, WREP), jnp.float32),
        pltpu.VMEM((TPW // 2, WREP), jnp.float32),
        pltpu.SemaphoreType.DMA,
        pltpu.SemaphoreType.DMA,
        pltpu.SemaphoreType.DMA,
        pltpu.SemaphoreType.DMA,
    ],
)
def _sc_scatter(x_hbm, idx_hbm, w0_hbm, w1_hbm, xs_hbm, ws_hbm,
                idx0_v, idx1_v, rows_v, wr0_v, wr1_v, s1, s2, s3, s4):
    wid = lax.axis_index("s") * 2 + lax.axis_index("c")
    base = wid * TPW
    half = TPW // 2
    pltpu.sync_copy(idx_hbm.at[wid, 0], idx0_v)
    pltpu.sync_copy(idx_hbm.at[wid, 1], idx1_v)
    for ch in range(2):
        cb = base + ch * half
        pltpu.sync_copy(x_hbm.at[pl.ds(cb, half)], rows_v)
        pltpu.sync_copy(w0_hbm.at[pl.ds(cb, half)], wr0_v)
        pltpu.sync_copy(w1_hbm.at[pl.ds(cb, half)], wr1_v)
        c1 = pltpu.async_copy(rows_v, xs_hbm.at[idx0_v.at[ch]], s1)
        c2 = pltpu.async_copy(rows_v, xs_hbm.at[idx1_v.at[ch]], s2)
        c3 = pltpu.async_copy(wr0_v, ws_hbm.at[idx0_v.at[ch]], s3)
        c4 = pltpu.async_copy(wr1_v, ws_hbm.at[idx1_v.at[ch]], s4)
        c1.wait()
        c2.wait()
        c3.wait()
        c4.wait()


def _ffn_kernel(eob_ref, xs_ref, ws_ref, w1_ref, b1_ref, w2_ref, b2_ref,
                ys_ref):
    xv = xs_ref[...]
    h = jnp.dot(xv, w1_ref[0], preferred_element_type=jnp.float32)
    h = h + b1_ref[0]
    h = 0.5 * h * (1.0 + jax.lax.erf(h * _SQRT_HALF))
    y = jnp.dot(h, w2_ref[0], preferred_element_type=jnp.float32)
    y = y + b2_ref[0]
    ys_ref[...] = y * ws_ref[:, 0:1]


def _ffn(xs, ws, eob, W1, b1, W2, b2):
    grid_spec = pltpu.PrefetchScalarGridSpec(
        num_scalar_prefetch=1,
        grid=(G,),
        in_specs=[
            pl.BlockSpec((BLK_S, D_IN), lambda g, eob: (g, 0)),
            pl.BlockSpec((BLK_S, WREP), lambda g, eob: (g, 0)),
            pl.BlockSpec((1, D_IN, HIDDEN), lambda g, eob: (eob[g], 0, 0)),
            pl.BlockSpec((1, 1, HIDDEN), lambda g, eob: (eob[g], 0, 0)),
            pl.BlockSpec((1, HIDDEN, D_IN), lambda g, eob: (eob[g], 0, 0)),
            pl.BlockSpec((1, 1, D_IN), lambda g, eob: (eob[g], 0, 0)),
        ],
        out_specs=pl.BlockSpec((BLK_S, D_IN), lambda g, eob: (g, 0)),
    )
    return pl.pallas_call(
        _ffn_kernel,
        grid_spec=grid_spec,
        out_shape=jax.ShapeDtypeStruct((P, D_IN), jnp.float32),
    )(eob, xs, ws, W1, b1.reshape(E, 1, HIDDEN), W2, b2.reshape(E, 1, D_IN))


_HCH = TPW // 2


@functools.partial(
    pl.kernel,
    mesh=_SC_MESH,
    out_type=jax.ShapeDtypeStruct((N_TOK, D_IN), jnp.float32),
    scratch_types=[
        pltpu.VMEM((2, _HCH), jnp.int32),
        pltpu.VMEM((2, _HCH), jnp.int32),
        pltpu.VMEM((_HCH, D_IN), jnp.float32),
        pltpu.VMEM((_HCH, D_IN), jnp.float32),
        pltpu.SemaphoreType.DMA,
        pltpu.SemaphoreType.DMA,
    ],
)
def _sc_combine(ys_hbm, idx_hbm, out_hbm, idx0_v, idx1_v, g0, g1, s1, s2):
    wid = lax.axis_index("s") * 2 + lax.axis_index("c")
    base = wid * TPW
    pltpu.sync_copy(idx_hbm.at[wid, 0], idx0_v)
    pltpu.sync_copy(idx_hbm.at[wid, 1], idx1_v)
    for ch in range(2):
        off = ch * _HCH
        c1 = pltpu.async_copy(ys_hbm.at[idx0_v.at[ch]], g0, s1)
        c2 = pltpu.async_copy(ys_hbm.at[idx1_v.at[ch]], g1, s2)
        c1.wait()
        c2.wait()

        def tok_body(t, _):
            def f_body(k, _):
                a = g0[t, pl.ds(k * 16, 16)]
                b = g1[t, pl.ds(k * 16, 16)]
                g0[t, pl.ds(k * 16, 16)] = a + b
                return 0

            lax.fori_loop(0, D_IN // 16, f_body, 0)
            return 0

        lax.fori_loop(0, _HCH, tok_body, 0)
        pltpu.sync_copy(g0, out_hbm.at[pl.ds(base + off, _HCH)])


@jax.jit
def kernel(x, Wr, br, W1, b1, W2, b2):
    pos0, pos1, w0, w1, eob, aux = _router(x, Wr, br)
    idx_hbm = jnp.stack(
        [pos0.reshape(NW, 2, TPW // 2), pos1.reshape(NW, 2, TPW // 2)],
        axis=1)
    xs, ws = _sc_scatter(x, idx_hbm, w0, w1)
    ys = _ffn(xs, ws, eob.reshape(64)[:G], W1, b1, W2, b2)
    out = _sc_combine(ys, idx_hbm)
    return out, aux.reshape(())


# speedup vs baseline: 1.1572x; 1.1572x over previous
import functools

import jax
import jax.numpy as jnp
import numpy as np
from jax import lax
from jax.experimental import pallas as pl
from jax.experimental.pallas import tpu as pltpu
from jax.experimental.pallas import tpu_sc as plsc

D_IN = 768
E = 8
HIDDEN = 256
N_TOK = 4096
EPS = 0.1
_SQRT_HALF = 0.7071067811865476

BLK_S = 256
G = N_TOK * 2 // BLK_S + E
P = G * BLK_S
CHUNK = 512
NW = 32
TPW = N_TOK // NW
WREP = 128


def _router_kernel(x_ref, wr_ref, br_ref, pos0_ref, pos1_ref, w0_ref,
                   w1_ref, eob_ref, aux_ref):
    x = x_ref[...]
    logits = jnp.dot(x, wr_ref[...], preferred_element_type=jnp.float32)
    logits = logits + br_ref[...]
    m = jnp.max(logits, axis=-1, keepdims=True)
    ex = jnp.exp(logits - m)
    probs = ex / jnp.sum(ex, axis=-1, keepdims=True)
    probs = (1.0 - EPS) * probs + EPS / E

    iota = jax.lax.broadcasted_iota(jnp.int32, probs.shape, 1)
    m1 = jnp.max(probs, axis=-1, keepdims=True)
    e1 = jnp.min(jnp.where(probs == m1, iota, E), axis=-1, keepdims=True)
    probs2 = jnp.where(iota == e1, -jnp.inf, probs)
    m2 = jnp.max(probs2, axis=-1, keepdims=True)
    e2 = jnp.min(jnp.where(probs2 == m2, iota, E), axis=-1, keepdims=True)

    load = jnp.sum(probs, axis=0, keepdims=True) / N_TOK
    aux = jnp.sum(load * jnp.log(load * E + 1e-9)) / np.log(E + 1e-9)
    aux_ref[...] = jnp.reshape(aux, (1, 1))

    oh1 = (iota == e1).astype(jnp.float32)
    oh2 = (iota == e2).astype(jnp.float32)
    C = oh1 + oh2

    r = jax.lax.broadcasted_iota(jnp.int32, (CHUNK, CHUNK), 0)
    c = jax.lax.broadcasted_iota(jnp.int32, (CHUNK, CHUNK), 1)
    T = (r > c).astype(jnp.float32)
    run = jnp.zeros((1, E), jnp.float32)
    chunks = []
    for b in range(N_TOK // CHUNK):
        Cb = C[b * CHUNK:(b + 1) * CHUNK]
        chunks.append(jnp.dot(T, Cb, preferred_element_type=jnp.float32) + run)
        run = run + jnp.sum(Cb, axis=0, keepdims=True)
    Pm = jnp.concatenate(chunks, axis=0)

    padded = jnp.ceil(run / BLK_S) * BLK_S
    ri = jax.lax.broadcasted_iota(jnp.int32, (E, E), 0)
    ci = jax.lax.broadcasted_iota(jnp.int32, (E, E), 1)
    TU = (ri < ci).astype(jnp.float32)
    ps = jnp.dot(padded, TU, preferred_element_type=jnp.float32)

    base = ps + Pm
    pos0 = jnp.sum(oh1 * base, axis=1, keepdims=True)
    pos1 = jnp.sum(oh2 * base, axis=1, keepdims=True)
    pos0_ref[...] = pos0.astype(jnp.int32)
    pos1_ref[...] = pos1.astype(jnp.int32)
    w0_ref[...] = jnp.broadcast_to(m1, (N_TOK, WREP))
    w1_ref[...] = jnp.broadcast_to(m2, (N_TOK, WREP))

    pad_end = ps + padded
    gi = jax.lax.broadcasted_iota(
        jnp.int32, (1, 64), 1).astype(jnp.float32) * BLK_S
    acc = jnp.zeros((1, 64), jnp.float32)
    for e in range(E):
        acc = acc + (gi >= pad_end[0:1, e:e + 1]).astype(jnp.float32)
    eob_ref[...] = jnp.minimum(acc, E - 1).astype(jnp.int32)


def _router(x, Wr, br):
    return pl.pallas_call(
        _router_kernel,
        grid=(1,),
        in_specs=[
            pl.BlockSpec((N_TOK, D_IN), lambda g: (0, 0)),
            pl.BlockSpec((D_IN, E), lambda g: (0, 0)),
            pl.BlockSpec((E,), lambda g: (0,)),
        ],
        out_specs=[
            pl.BlockSpec((N_TOK, 1), lambda g: (0, 0)),
            pl.BlockSpec((N_TOK, 1), lambda g: (0, 0)),
            pl.BlockSpec((N_TOK, WREP), lambda g: (0, 0)),
            pl.BlockSpec((N_TOK, WREP), lambda g: (0, 0)),
            pl.BlockSpec((1, 64), lambda g: (0, 0)),
            pl.BlockSpec((1, 1), lambda g: (0, 0)),
        ],
        out_shape=[
            jax.ShapeDtypeStruct((N_TOK, 1), jnp.int32),
            jax.ShapeDtypeStruct((N_TOK, 1), jnp.int32),
            jax.ShapeDtypeStruct((N_TOK, WREP), jnp.float32),
            jax.ShapeDtypeStruct((N_TOK, WREP), jnp.float32),
            jax.ShapeDtypeStruct((1, 64), jnp.int32),
            jax.ShapeDtypeStruct((1, 1), jnp.float32),
        ],
    )(x, Wr, br)


_SC_MESH = plsc.VectorSubcoreMesh(core_axis_name="c", subcore_axis_name="s")


@functools.partial(
    pl.kernel,
    mesh=_SC_MESH,
    out_type=[
        jax.ShapeDtypeStruct((P, D_IN), jnp.float32),
        jax.ShapeDtypeStruct((P, WREP), jnp.float32),
    ],
    scratch_types=[
        pltpu.VMEM((2, TPW // 2), jnp.int32),
        pltpu.VMEM((2, TPW // 2), jnp.int32),
        pltpu.VMEM((TPW // 2, D_IN), jnp.float32),
        pltpu.VMEM((TPW // 2, WREP), jnp.float32),
        pltpu.VMEM((TPW // 2, WREP), jnp.float32),
        pltpu.SemaphoreType.DMA,
        pltpu.SemaphoreType.DMA,
        pltpu.SemaphoreType.DMA,
        pltpu.SemaphoreType.DMA,
    ],
)
def _sc_scatter(x_hbm, idx_hbm, w0_hbm, w1_hbm, xs_hbm, ws_hbm,
                idx0_v, idx1_v, rows_v, wr0_v, wr1_v, s1, s2, s3, s4):
    wid = lax.axis_index("s") * 2 + lax.axis_index("c")
    base = wid * TPW
    half = TPW // 2
    pltpu.sync_copy(idx_hbm.at[wid, 0], idx0_v)
    pltpu.sync_copy(idx_hbm.at[wid, 1], idx1_v)
    for ch in range(2):
        cb = base + ch * half
        pltpu.sync_copy(x_hbm.at[pl.ds(cb, half)], rows_v)
        pltpu.sync_copy(w0_hbm.at[pl.ds(cb, half)], wr0_v)
        pltpu.sync_copy(w1_hbm.at[pl.ds(cb, half)], wr1_v)
        c1 = pltpu.async_copy(rows_v, xs_hbm.at[idx0_v.at[ch]], s1)
        c2 = pltpu.async_copy(rows_v, xs_hbm.at[idx1_v.at[ch]], s2)
        c3 = pltpu.async_copy(wr0_v, ws_hbm.at[idx0_v.at[ch]], s3)
        c4 = pltpu.async_copy(wr1_v, ws_hbm.at[idx1_v.at[ch]], s4)
        c1.wait()
        c2.wait()
        c3.wait()
        c4.wait()


def _ffn_kernel(eob_ref, xs_ref, ws_ref, w1_ref, b1_ref, w2_ref, b2_ref,
                ys_ref):
    xv = xs_ref[...]
    h = jnp.dot(xv, w1_ref[0], preferred_element_type=jnp.float32)
    h = h + b1_ref[0]
    h = 0.5 * h * (1.0 + jax.lax.erf(h * _SQRT_HALF))
    y = jnp.dot(h, w2_ref[0], preferred_element_type=jnp.float32)
    y = y + b2_ref[0]
    ys_ref[...] = y * ws_ref[:, 0:1]


def _ffn(xs, ws, eob, W1, b1, W2, b2):
    grid_spec = pltpu.PrefetchScalarGridSpec(
        num_scalar_prefetch=1,
        grid=(G,),
        in_specs=[
            pl.BlockSpec((BLK_S, D_IN), lambda g, eob: (g, 0)),
            pl.BlockSpec((BLK_S, WREP), lambda g, eob: (g, 0)),
            pl.BlockSpec((1, D_IN, HIDDEN), lambda g, eob: (eob[g], 0, 0)),
            pl.BlockSpec((1, 1, HIDDEN), lambda g, eob: (eob[g], 0, 0)),
            pl.BlockSpec((1, HIDDEN, D_IN), lambda g, eob: (eob[g], 0, 0)),
            pl.BlockSpec((1, 1, D_IN), lambda g, eob: (eob[g], 0, 0)),
        ],
        out_specs=pl.BlockSpec((BLK_S, D_IN), lambda g, eob: (g, 0)),
    )
    return pl.pallas_call(
        _ffn_kernel,
        grid_spec=grid_spec,
        out_shape=jax.ShapeDtypeStruct((P, D_IN), jnp.float32),
    )(eob, xs, ws, W1, b1.reshape(E, 1, HIDDEN), W2, b2.reshape(E, 1, D_IN))


_HCH = TPW // 2


@functools.partial(
    pl.kernel,
    mesh=_SC_MESH,
    out_type=jax.ShapeDtypeStruct((N_TOK, D_IN), jnp.float32),
    scratch_types=[
        pltpu.VMEM((2, _HCH), jnp.int32),
        pltpu.VMEM((2, _HCH), jnp.int32),
        pltpu.VMEM((_HCH, D_IN), jnp.float32),
        pltpu.VMEM((_HCH, D_IN), jnp.float32),
        pltpu.SemaphoreType.DMA,
        pltpu.SemaphoreType.DMA,
    ],
)
def _sc_combine(ys_hbm, idx_hbm, out_hbm, idx0_v, idx1_v, g0, g1, s1, s2):
    wid = lax.axis_index("s") * 2 + lax.axis_index("c")
    base = wid * TPW
    pltpu.sync_copy(idx_hbm.at[wid, 0], idx0_v)
    pltpu.sync_copy(idx_hbm.at[wid, 1], idx1_v)
    for ch in range(2):
        off = ch * _HCH
        c1 = pltpu.async_copy(ys_hbm.at[idx0_v.at[ch]], g0, s1)
        c2 = pltpu.async_copy(ys_hbm.at[idx1_v.at[ch]], g1, s2)
        c1.wait()
        c2.wait()

        def tok_body(t, _):
            for k in range(D_IN // 16):
                a = g0[t, pl.ds(k * 16, 16)]
                b = g1[t, pl.ds(k * 16, 16)]
                g0[t, pl.ds(k * 16, 16)] = a + b
            return 0

        lax.fori_loop(0, _HCH, tok_body, 0)
        pltpu.sync_copy(g0, out_hbm.at[pl.ds(base + off, _HCH)])


@jax.jit
def kernel(x, Wr, br, W1, b1, W2, b2):
    pos0, pos1, w0, w1, eob, aux = _router(x, Wr, br)
    idx_hbm = jnp.stack(
        [pos0.reshape(NW, 2, TPW // 2), pos1.reshape(NW, 2, TPW // 2)],
        axis=1)
    xs, ws = _sc_scatter(x, idx_hbm, w0, w1)
    ys = _ffn(xs, ws, eob.reshape(64)[:G], W1, b1, W2, b2)
    out = _sc_combine(ys, idx_hbm)
    return out, aux.reshape(())


# speedup vs baseline: 1.2109x; 1.0464x over previous
import functools

import jax
import jax.numpy as jnp
import numpy as np
from jax import lax
from jax.experimental import pallas as pl
from jax.experimental.pallas import tpu as pltpu
from jax.experimental.pallas import tpu_sc as plsc

D_IN = 768
E = 8
HIDDEN = 256
N_TOK = 4096
EPS = 0.1
_SQRT_HALF = 0.7071067811865476

BLK_S = 512
G = N_TOK * 2 // BLK_S + E
P = G * BLK_S
CHUNK = 512
NW = 32
TPW = N_TOK // NW
WREP = 128


def _router_kernel(x_ref, wr_ref, br_ref, pos0_ref, pos1_ref, w0_ref,
                   w1_ref, eob_ref, aux_ref):
    x = x_ref[...]
    logits = jnp.dot(x, wr_ref[...], preferred_element_type=jnp.float32)
    logits = logits + br_ref[...]
    m = jnp.max(logits, axis=-1, keepdims=True)
    ex = jnp.exp(logits - m)
    probs = ex / jnp.sum(ex, axis=-1, keepdims=True)
    probs = (1.0 - EPS) * probs + EPS / E

    iota = jax.lax.broadcasted_iota(jnp.int32, probs.shape, 1)
    m1 = jnp.max(probs, axis=-1, keepdims=True)
    e1 = jnp.min(jnp.where(probs == m1, iota, E), axis=-1, keepdims=True)
    probs2 = jnp.where(iota == e1, -jnp.inf, probs)
    m2 = jnp.max(probs2, axis=-1, keepdims=True)
    e2 = jnp.min(jnp.where(probs2 == m2, iota, E), axis=-1, keepdims=True)

    load = jnp.sum(probs, axis=0, keepdims=True) / N_TOK
    aux = jnp.sum(load * jnp.log(load * E + 1e-9)) / np.log(E + 1e-9)
    aux_ref[...] = jnp.reshape(aux, (1, 1))

    oh1 = (iota == e1).astype(jnp.float32)
    oh2 = (iota == e2).astype(jnp.float32)
    C = oh1 + oh2

    r = jax.lax.broadcasted_iota(jnp.int32, (CHUNK, CHUNK), 0)
    c = jax.lax.broadcasted_iota(jnp.int32, (CHUNK, CHUNK), 1)
    T = (r > c).astype(jnp.float32)
    run = jnp.zeros((1, E), jnp.float32)
    chunks = []
    for b in range(N_TOK // CHUNK):
        Cb = C[b * CHUNK:(b + 1) * CHUNK]
        chunks.append(jnp.dot(T, Cb, preferred_element_type=jnp.float32) + run)
        run = run + jnp.sum(Cb, axis=0, keepdims=True)
    Pm = jnp.concatenate(chunks, axis=0)

    padded = jnp.ceil(run / BLK_S) * BLK_S
    ri = jax.lax.broadcasted_iota(jnp.int32, (E, E), 0)
    ci = jax.lax.broadcasted_iota(jnp.int32, (E, E), 1)
    TU = (ri < ci).astype(jnp.float32)
    ps = jnp.dot(padded, TU, preferred_element_type=jnp.float32)

    base = ps + Pm
    pos0 = jnp.sum(oh1 * base, axis=1, keepdims=True)
    pos1 = jnp.sum(oh2 * base, axis=1, keepdims=True)
    pos0_ref[...] = pos0.astype(jnp.int32)
    pos1_ref[...] = pos1.astype(jnp.int32)
    w0_ref[...] = jnp.broadcast_to(m1, (N_TOK, WREP))
    w1_ref[...] = jnp.broadcast_to(m2, (N_TOK, WREP))

    pad_end = ps + padded
    gi = jax.lax.broadcasted_iota(
        jnp.int32, (1, 64), 1).astype(jnp.float32) * BLK_S
    acc = jnp.zeros((1, 64), jnp.float32)
    for e in range(E):
        acc = acc + (gi >= pad_end[0:1, e:e + 1]).astype(jnp.float32)
    eob_ref[...] = jnp.minimum(acc, E - 1).astype(jnp.int32)


def _router(x, Wr, br):
    return pl.pallas_call(
        _router_kernel,
        grid=(1,),
        in_specs=[
            pl.BlockSpec((N_TOK, D_IN), lambda g: (0, 0)),
            pl.BlockSpec((D_IN, E), lambda g: (0, 0)),
            pl.BlockSpec((E,), lambda g: (0,)),
        ],
        out_specs=[
            pl.BlockSpec((N_TOK, 1), lambda g: (0, 0)),
            pl.BlockSpec((N_TOK, 1), lambda g: (0, 0)),
            pl.BlockSpec((N_TOK, WREP), lambda g: (0, 0)),
            pl.BlockSpec((N_TOK, WREP), lambda g: (0, 0)),
            pl.BlockSpec((1, 64), lambda g: (0, 0)),
            pl.BlockSpec((1, 1), lambda g: (0, 0)),
        ],
        out_shape=[
            jax.ShapeDtypeStruct((N_TOK, 1), jnp.int32),
            jax.ShapeDtypeStruct((N_TOK, 1), jnp.int32),
            jax.ShapeDtypeStruct((N_TOK, WREP), jnp.float32),
            jax.ShapeDtypeStruct((N_TOK, WREP), jnp.float32),
            jax.ShapeDtypeStruct((1, 64), jnp.int32),
            jax.ShapeDtypeStruct((1, 1), jnp.float32),
        ],
    )(x, Wr, br)


_SC_MESH = plsc.VectorSubcoreMesh(core_axis_name="c", subcore_axis_name="s")


@functools.partial(
    pl.kernel,
    mesh=_SC_MESH,
    out_type=[
        jax.ShapeDtypeStruct((P, D_IN), jnp.float32),
        jax.ShapeDtypeStruct((P, WREP), jnp.float32),
    ],
    scratch_types=[
        pltpu.VMEM((2, TPW // 2), jnp.int32),
        pltpu.VMEM((2, TPW // 2), jnp.int32),
        pltpu.VMEM((TPW // 2, D_IN), jnp.float32),
        pltpu.VMEM((TPW // 2, WREP), jnp.float32),
        pltpu.VMEM((TPW // 2, WREP), jnp.float32),
        pltpu.SemaphoreType.DMA,
        pltpu.SemaphoreType.DMA,
        pltpu.SemaphoreType.DMA,
        pltpu.SemaphoreType.DMA,
    ],
)
def _sc_scatter(x_hbm, idx_hbm, w0_hbm, w1_hbm, xs_hbm, ws_hbm,
                idx0_v, idx1_v, rows_v, wr0_v, wr1_v, s1, s2, s3, s4):
    wid = lax.axis_index("s") * 2 + lax.axis_index("c")
    base = wid * TPW
    half = TPW // 2
    pltpu.sync_copy(idx_hbm.at[wid, 0], idx0_v)
    pltpu.sync_copy(idx_hbm.at[wid, 1], idx1_v)
    for ch in range(2):
        cb = base + ch * half
        pltpu.sync_copy(x_hbm.at[pl.ds(cb, half)], rows_v)
        pltpu.sync_copy(w0_hbm.at[pl.ds(cb, half)], wr0_v)
        pltpu.sync_copy(w1_hbm.at[pl.ds(cb, half)], wr1_v)
        c1 = pltpu.async_copy(rows_v, xs_hbm.at[idx0_v.at[ch]], s1)
        c2 = pltpu.async_copy(rows_v, xs_hbm.at[idx1_v.at[ch]], s2)
        c3 = pltpu.async_copy(wr0_v, ws_hbm.at[idx0_v.at[ch]], s3)
        c4 = pltpu.async_copy(wr1_v, ws_hbm.at[idx1_v.at[ch]], s4)
        c1.wait()
        c2.wait()
        c3.wait()
        c4.wait()


def _ffn_kernel(eob_ref, xs_ref, ws_ref, w1_ref, b1_ref, w2_ref, b2_ref,
                ys_ref):
    xv = xs_ref[...]
    h = jnp.dot(xv, w1_ref[0], preferred_element_type=jnp.float32)
    h = h + b1_ref[0]
    h = 0.5 * h * (1.0 + jax.lax.erf(h * _SQRT_HALF))
    y = jnp.dot(h, w2_ref[0], preferred_element_type=jnp.float32)
    y = y + b2_ref[0]
    ys_ref[...] = y * ws_ref[:, 0:1]


def _ffn(xs, ws, eob, W1, b1, W2, b2):
    grid_spec = pltpu.PrefetchScalarGridSpec(
        num_scalar_prefetch=1,
        grid=(G,),
        in_specs=[
            pl.BlockSpec((BLK_S, D_IN), lambda g, eob: (g, 0)),
            pl.BlockSpec((BLK_S, WREP), lambda g, eob: (g, 0)),
            pl.BlockSpec((1, D_IN, HIDDEN), lambda g, eob: (eob[g], 0, 0)),
            pl.BlockSpec((1, 1, HIDDEN), lambda g, eob: (eob[g], 0, 0)),
            pl.BlockSpec((1, HIDDEN, D_IN), lambda g, eob: (eob[g], 0, 0)),
            pl.BlockSpec((1, 1, D_IN), lambda g, eob: (eob[g], 0, 0)),
        ],
        out_specs=pl.BlockSpec((BLK_S, D_IN), lambda g, eob: (g, 0)),
    )
    return pl.pallas_call(
        _ffn_kernel,
        grid_spec=grid_spec,
        out_shape=jax.ShapeDtypeStruct((P, D_IN), jnp.float32),
    )(eob, xs, ws, W1, b1.reshape(E, 1, HIDDEN), W2, b2.reshape(E, 1, D_IN))


_HCH = TPW // 2


@functools.partial(
    pl.kernel,
    mesh=_SC_MESH,
    out_type=jax.ShapeDtypeStruct((N_TOK, D_IN), jnp.float32),
    scratch_types=[
        pltpu.VMEM((2, _HCH), jnp.int32),
        pltpu.VMEM((2, _HCH), jnp.int32),
        pltpu.VMEM((_HCH, D_IN), jnp.float32),
        pltpu.VMEM((_HCH, D_IN), jnp.float32),
        pltpu.SemaphoreType.DMA,
        pltpu.SemaphoreType.DMA,
    ],
)
def _sc_combine(ys_hbm, idx_hbm, out_hbm, idx0_v, idx1_v, g0, g1, s1, s2):
    wid = lax.axis_index("s") * 2 + lax.axis_index("c")
    base = wid * TPW
    pltpu.sync_copy(idx_hbm.at[wid, 0], idx0_v)
    pltpu.sync_copy(idx_hbm.at[wid, 1], idx1_v)
    for ch in range(2):
        off = ch * _HCH
        c1 = pltpu.async_copy(ys_hbm.at[idx0_v.at[ch]], g0, s1)
        c2 = pltpu.async_copy(ys_hbm.at[idx1_v.at[ch]], g1, s2)
        c1.wait()
        c2.wait()

        def tok_body(t, _):
            for k in range(D_IN // 16):
                a = g0[t, pl.ds(k * 16, 16)]
                b = g1[t, pl.ds(k * 16, 16)]
                g0[t, pl.ds(k * 16, 16)] = a + b
            return 0

        lax.fori_loop(0, _HCH, tok_body, 0)
        pltpu.sync_copy(g0, out_hbm.at[pl.ds(base + off, _HCH)])


@jax.jit
def kernel(x, Wr, br, W1, b1, W2, b2):
    pos0, pos1, w0, w1, eob, aux = _router(x, Wr, br)
    idx_hbm = jnp.stack(
        [pos0.reshape(NW, 2, TPW // 2), pos1.reshape(NW, 2, TPW // 2)],
        axis=1)
    xs, ws = _sc_scatter(x, idx_hbm, w0, w1)
    ys = _ffn(xs, ws, eob.reshape(64)[:G], W1, b1, W2, b2)
    out = _sc_combine(ys, idx_hbm)
    return out, aux.reshape(())


# speedup vs baseline: 1.2406x; 1.0246x over previous
import functools

import jax
import jax.numpy as jnp
import numpy as np
from jax import lax
from jax.experimental import pallas as pl
from jax.experimental.pallas import tpu as pltpu
from jax.experimental.pallas import tpu_sc as plsc

D_IN = 768
E = 8
HIDDEN = 256
N_TOK = 4096
EPS = 0.1
_SQRT_HALF = 0.7071067811865476

BLK_S = 512
G = N_TOK * 2 // BLK_S + E
P = G * BLK_S
CHUNK = 512
NW = 32
TPW = N_TOK // NW
WREP = 128


def _router_kernel(x_ref, wr_ref, br_ref, pos0_ref, pos1_ref, w0_ref,
                   w1_ref, eob_ref, aux_ref):
    x = x_ref[...]
    logits = jnp.dot(x, wr_ref[...], preferred_element_type=jnp.float32)
    logits = logits + br_ref[...]
    m = jnp.max(logits, axis=-1, keepdims=True)
    ex = jnp.exp(logits - m)
    probs = ex / jnp.sum(ex, axis=-1, keepdims=True)
    probs = (1.0 - EPS) * probs + EPS / E

    iota = jax.lax.broadcasted_iota(jnp.int32, probs.shape, 1)
    m1 = jnp.max(probs, axis=-1, keepdims=True)
    e1 = jnp.min(jnp.where(probs == m1, iota, E), axis=-1, keepdims=True)
    probs2 = jnp.where(iota == e1, -jnp.inf, probs)
    m2 = jnp.max(probs2, axis=-1, keepdims=True)
    e2 = jnp.min(jnp.where(probs2 == m2, iota, E), axis=-1, keepdims=True)

    load = jnp.sum(probs, axis=0, keepdims=True) / N_TOK
    aux = jnp.sum(load * jnp.log(load * E + 1e-9)) / np.log(E + 1e-9)
    aux_ref[...] = jnp.reshape(aux, (1, 1))

    oh1 = (iota == e1).astype(jnp.float32)
    oh2 = (iota == e2).astype(jnp.float32)
    C = oh1 + oh2

    r = jax.lax.broadcasted_iota(jnp.int32, (CHUNK, CHUNK), 0)
    c = jax.lax.broadcasted_iota(jnp.int32, (CHUNK, CHUNK), 1)
    T = (r > c).astype(jnp.float32)
    run = jnp.zeros((1, E), jnp.float32)
    chunks = []
    for b in range(N_TOK // CHUNK):
        Cb = C[b * CHUNK:(b + 1) * CHUNK]
        chunks.append(jnp.dot(T, Cb, preferred_element_type=jnp.float32) + run)
        run = run + jnp.sum(Cb, axis=0, keepdims=True)
    Pm = jnp.concatenate(chunks, axis=0)

    padded = jnp.ceil(run / BLK_S) * BLK_S
    ri = jax.lax.broadcasted_iota(jnp.int32, (E, E), 0)
    ci = jax.lax.broadcasted_iota(jnp.int32, (E, E), 1)
    TU = (ri < ci).astype(jnp.float32)
    ps = jnp.dot(padded, TU, preferred_element_type=jnp.float32)

    base = ps + Pm
    pos0 = jnp.sum(oh1 * base, axis=1, keepdims=True)
    pos1 = jnp.sum(oh2 * base, axis=1, keepdims=True)
    pos0_ref[...] = pos0.astype(jnp.int32)
    pos1_ref[...] = pos1.astype(jnp.int32)
    w0_ref[...] = jnp.broadcast_to(m1, (N_TOK, WREP))
    w1_ref[...] = jnp.broadcast_to(m2, (N_TOK, WREP))

    pad_end = ps + padded
    gi = jax.lax.broadcasted_iota(
        jnp.int32, (1, 64), 1).astype(jnp.float32) * BLK_S
    acc = jnp.zeros((1, 64), jnp.float32)
    for e in range(E):
        acc = acc + (gi >= pad_end[0:1, e:e + 1]).astype(jnp.float32)
    eob_ref[...] = jnp.minimum(acc, E - 1).astype(jnp.int32)


def _router(x, Wr, br):
    return pl.pallas_call(
        _router_kernel,
        grid=(1,),
        in_specs=[
            pl.BlockSpec((N_TOK, D_IN), lambda g: (0, 0)),
            pl.BlockSpec((D_IN, E), lambda g: (0, 0)),
            pl.BlockSpec((E,), lambda g: (0,)),
        ],
        out_specs=[
            pl.BlockSpec((N_TOK, 1), lambda g: (0, 0)),
            pl.BlockSpec((N_TOK, 1), lambda g: (0, 0)),
            pl.BlockSpec((N_TOK, WREP), lambda g: (0, 0)),
            pl.BlockSpec((N_TOK, WREP), lambda g: (0, 0)),
            pl.BlockSpec((1, 64), lambda g: (0, 0)),
            pl.BlockSpec((1, 1), lambda g: (0, 0)),
        ],
        out_shape=[
            jax.ShapeDtypeStruct((N_TOK, 1), jnp.int32),
            jax.ShapeDtypeStruct((N_TOK, 1), jnp.int32),
            jax.ShapeDtypeStruct((N_TOK, WREP), jnp.float32),
            jax.ShapeDtypeStruct((N_TOK, WREP), jnp.float32),
            jax.ShapeDtypeStruct((1, 64), jnp.int32),
            jax.ShapeDtypeStruct((1, 1), jnp.float32),
        ],
    )(x, Wr, br)


_SC_MESH = plsc.VectorSubcoreMesh(core_axis_name="c", subcore_axis_name="s")


@functools.partial(
    pl.kernel,
    mesh=_SC_MESH,
    out_type=[
        jax.ShapeDtypeStruct((P, D_IN), jnp.float32),
        jax.ShapeDtypeStruct((P, WREP), jnp.float32),
    ],
    scratch_types=[
        pltpu.VMEM((2, TPW // 2), jnp.int32),
        pltpu.VMEM((2, TPW // 2), jnp.int32),
        pltpu.VMEM((TPW // 2, D_IN), jnp.float32),
        pltpu.VMEM((TPW // 2, WREP), jnp.float32),
        pltpu.VMEM((TPW // 2, WREP), jnp.float32),
        pltpu.SemaphoreType.DMA,
        pltpu.SemaphoreType.DMA,
        pltpu.SemaphoreType.DMA,
        pltpu.SemaphoreType.DMA,
    ],
)
def _sc_scatter(x_hbm, idx_hbm, w0_hbm, w1_hbm, xs_hbm, ws_hbm,
                idx0_v, idx1_v, rows_v, wr0_v, wr1_v, s1, s2, s3, s4):
    wid = lax.axis_index("s") * 2 + lax.axis_index("c")
    base = wid * TPW
    half = TPW // 2
    pltpu.sync_copy(idx_hbm.at[wid, 0], idx0_v)
    pltpu.sync_copy(idx_hbm.at[wid, 1], idx1_v)
    for ch in range(2):
        cb = base + ch * half
        pltpu.sync_copy(x_hbm.at[pl.ds(cb, half)], rows_v)
        pltpu.sync_copy(w0_hbm.at[pl.ds(cb, half)], wr0_v)
        pltpu.sync_copy(w1_hbm.at[pl.ds(cb, half)], wr1_v)
        c1 = pltpu.async_copy(rows_v, xs_hbm.at[idx0_v.at[ch]], s1)
        c2 = pltpu.async_copy(rows_v, xs_hbm.at[idx1_v.at[ch]], s2)
        c3 = pltpu.async_copy(wr0_v, ws_hbm.at[idx0_v.at[ch]], s3)
        c4 = pltpu.async_copy(wr1_v, ws_hbm.at[idx1_v.at[ch]], s4)
        c1.wait()
        c2.wait()
        c3.wait()
        c4.wait()


def _ffn_kernel(eob_ref, xs_ref, ws_ref, w1_ref, b1_ref, w2_ref, b2_ref,
                ys_ref):
    xv = xs_ref[...]
    h = jnp.dot(xv, w1_ref[0], preferred_element_type=jnp.float32)
    h = h + b1_ref[0]
    h = 0.5 * h * (1.0 + jax.lax.erf(h * _SQRT_HALF))
    y = jnp.dot(h, w2_ref[0], preferred_element_type=jnp.float32)
    y = y + b2_ref[0]
    ys_ref[...] = y * ws_ref[:, 0:1]


def _ffn(xs, ws, eob, W1, b1, W2, b2):
    grid_spec = pltpu.PrefetchScalarGridSpec(
        num_scalar_prefetch=1,
        grid=(G,),
        in_specs=[
            pl.BlockSpec((BLK_S, D_IN), lambda g, eob: (g, 0)),
            pl.BlockSpec((BLK_S, WREP), lambda g, eob: (g, 0)),
            pl.BlockSpec((1, D_IN, HIDDEN), lambda g, eob: (eob[g], 0, 0)),
            pl.BlockSpec((1, 1, HIDDEN), lambda g, eob: (eob[g], 0, 0)),
            pl.BlockSpec((1, HIDDEN, D_IN), lambda g, eob: (eob[g], 0, 0)),
            pl.BlockSpec((1, 1, D_IN), lambda g, eob: (eob[g], 0, 0)),
        ],
        out_specs=pl.BlockSpec((BLK_S, D_IN), lambda g, eob: (g, 0)),
    )
    return pl.pallas_call(
        _ffn_kernel,
        grid_spec=grid_spec,
        out_shape=jax.ShapeDtypeStruct((P, D_IN), jnp.float32),
    )(eob, xs, ws, W1, b1.reshape(E, 1, HIDDEN), W2, b2.reshape(E, 1, D_IN))


_NCH = 4
_CW = TPW // _NCH


@functools.partial(
    pl.kernel,
    mesh=_SC_MESH,
    out_type=jax.ShapeDtypeStruct((N_TOK, D_IN), jnp.float32),
    scratch_types=[
        pltpu.VMEM((_NCH, _CW), jnp.int32),
        pltpu.VMEM((_NCH, _CW), jnp.int32),
        pltpu.VMEM((_CW, D_IN), jnp.float32),
        pltpu.VMEM((_CW, D_IN), jnp.float32),
        pltpu.VMEM((_CW, D_IN), jnp.float32),
        pltpu.VMEM((_CW, D_IN), jnp.float32),
        pltpu.SemaphoreType.DMA,
        pltpu.SemaphoreType.DMA,
        pltpu.SemaphoreType.DMA,
        pltpu.SemaphoreType.DMA,
        pltpu.SemaphoreType.DMA,
        pltpu.SemaphoreType.DMA,
    ],
)
def _sc_combine(ys_hbm, idx_hbm, out_hbm, idx0_v, idx1_v,
                a0, a1, b0, b1, ga0, ga1, gb0, gb1, st0, st1):
    wid = lax.axis_index("s") * 2 + lax.axis_index("c")
    base = wid * TPW
    abufs, bbufs = (a0, a1), (b0, b1)
    gas, gbs, sts = (ga0, ga1), (gb0, gb1), (st0, st1)
    pltpu.sync_copy(idx_hbm.at[wid, 0], idx0_v)
    pltpu.sync_copy(idx_hbm.at[wid, 1], idx1_v)
    pltpu.async_copy(ys_hbm.at[idx0_v.at[0]], a0, ga0)
    pltpu.async_copy(ys_hbm.at[idx1_v.at[0]], b0, gb0)
    for ch in range(_NCH):
        p = ch & 1
        if ch + 1 < _NCH:
            q = (ch + 1) & 1
            if ch + 1 >= 2:
                pltpu.make_async_copy(
                    abufs[q],
                    out_hbm.at[pl.ds(base + (ch - 1) * _CW, _CW)],
                    sts[q]).wait()
            pltpu.async_copy(ys_hbm.at[idx0_v.at[ch + 1]], abufs[q], gas[q])
            pltpu.async_copy(ys_hbm.at[idx1_v.at[ch + 1]], bbufs[q], gbs[q])
        pltpu.make_async_copy(ys_hbm.at[idx0_v.at[ch]], abufs[p], gas[p]).wait()
        pltpu.make_async_copy(ys_hbm.at[idx1_v.at[ch]], bbufs[p], gbs[p]).wait()
        ga, gb = abufs[p], bbufs[p]

        def tok_body(t, _, ga=ga, gb=gb):
            for k in range(D_IN // 16):
                ga[t, pl.ds(k * 16, 16)] = (
                    ga[t, pl.ds(k * 16, 16)] + gb[t, pl.ds(k * 16, 16)])
            return 0

        lax.fori_loop(0, _CW, tok_body, 0)
        pltpu.async_copy(
            abufs[p], out_hbm.at[pl.ds(base + ch * _CW, _CW)], sts[p])
    for ch in (_NCH - 2, _NCH - 1):
        p = ch & 1
        pltpu.make_async_copy(
            abufs[p], out_hbm.at[pl.ds(base + ch * _CW, _CW)], sts[p]).wait()


@jax.jit
def kernel(x, Wr, br, W1, b1, W2, b2):
    pos0, pos1, w0, w1, eob, aux = _router(x, Wr, br)
    idx_hbm = jnp.stack(
        [pos0.reshape(NW, 2, TPW // 2), pos1.reshape(NW, 2, TPW // 2)],
        axis=1)
    xs, ws = _sc_scatter(x, idx_hbm, w0, w1)
    ys = _ffn(xs, ws, eob.reshape(64)[:G], W1, b1, W2, b2)
    out = _sc_combine(ys, idx_hbm.reshape(NW, 2, _NCH, _CW))
    return out, aux.reshape(())


# speedup vs baseline: 1.2606x; 1.0161x over previous
import functools

import jax
import jax.numpy as jnp
import numpy as np
from jax import lax
from jax.experimental import pallas as pl
from jax.experimental.pallas import tpu as pltpu
from jax.experimental.pallas import tpu_sc as plsc

D_IN = 768
E = 8
HIDDEN = 256
N_TOK = 4096
EPS = 0.1
_SQRT_HALF = 0.7071067811865476

BLK_S = 512
G = N_TOK * 2 // BLK_S + E
P = G * BLK_S
CHUNK = 512
NW = 32
TPW = N_TOK // NW
WREP = 128


def _router_kernel(x_ref, wr_ref, br_ref, pos0_ref, pos1_ref, w0_ref,
                   w1_ref, eob_ref, aux_ref):
    x = x_ref[...]
    logits = jnp.dot(x, wr_ref[...], preferred_element_type=jnp.float32)
    logits = logits + br_ref[...]
    m = jnp.max(logits, axis=-1, keepdims=True)
    ex = jnp.exp(logits - m)
    probs = ex / jnp.sum(ex, axis=-1, keepdims=True)
    probs = (1.0 - EPS) * probs + EPS / E

    iota = jax.lax.broadcasted_iota(jnp.int32, probs.shape, 1)
    m1 = jnp.max(probs, axis=-1, keepdims=True)
    e1 = jnp.min(jnp.where(probs == m1, iota, E), axis=-1, keepdims=True)
    probs2 = jnp.where(iota == e1, -jnp.inf, probs)
    m2 = jnp.max(probs2, axis=-1, keepdims=True)
    e2 = jnp.min(jnp.where(probs2 == m2, iota, E), axis=-1, keepdims=True)

    load = jnp.sum(probs, axis=0, keepdims=True) / N_TOK
    aux = jnp.sum(load * jnp.log(load * E + 1e-9)) / np.log(E + 1e-9)
    aux_ref[...] = jnp.reshape(aux, (1, 1))

    oh1 = (iota == e1).astype(jnp.float32)
    oh2 = (iota == e2).astype(jnp.float32)
    C = oh1 + oh2

    r = jax.lax.broadcasted_iota(jnp.int32, (CHUNK, CHUNK), 0)
    c = jax.lax.broadcasted_iota(jnp.int32, (CHUNK, CHUNK), 1)
    T = (r > c).astype(jnp.float32)
    run = jnp.zeros((1, E), jnp.float32)
    chunks = []
    for b in range(N_TOK // CHUNK):
        Cb = C[b * CHUNK:(b + 1) * CHUNK]
        chunks.append(jnp.dot(T, Cb, preferred_element_type=jnp.float32) + run)
        run = run + jnp.sum(Cb, axis=0, keepdims=True)
    Pm = jnp.concatenate(chunks, axis=0)

    padded = jnp.ceil(run / BLK_S) * BLK_S
    ri = jax.lax.broadcasted_iota(jnp.int32, (E, E), 0)
    ci = jax.lax.broadcasted_iota(jnp.int32, (E, E), 1)
    TU = (ri < ci).astype(jnp.float32)
    ps = jnp.dot(padded, TU, preferred_element_type=jnp.float32)

    base = ps + Pm
    pos0 = jnp.sum(oh1 * base, axis=1, keepdims=True)
    pos1 = jnp.sum(oh2 * base, axis=1, keepdims=True)
    pos0_ref[...] = pos0.astype(jnp.int32)
    pos1_ref[...] = pos1.astype(jnp.int32)
    w0_ref[...] = jnp.broadcast_to(m1, (N_TOK, WREP))
    w1_ref[...] = jnp.broadcast_to(m2, (N_TOK, WREP))

    pad_end = ps + padded
    gi = jax.lax.broadcasted_iota(
        jnp.int32, (1, 64), 1).astype(jnp.float32) * BLK_S
    acc = jnp.zeros((1, 64), jnp.float32)
    for e in range(E):
        acc = acc + (gi >= pad_end[0:1, e:e + 1]).astype(jnp.float32)
    eob_ref[...] = jnp.minimum(acc, E - 1).astype(jnp.int32)


def _router(x, Wr, br):
    return pl.pallas_call(
        _router_kernel,
        grid=(1,),
        in_specs=[
            pl.BlockSpec((N_TOK, D_IN), lambda g: (0, 0)),
            pl.BlockSpec((D_IN, E), lambda g: (0, 0)),
            pl.BlockSpec((E,), lambda g: (0,)),
        ],
        out_specs=[
            pl.BlockSpec((N_TOK, 1), lambda g: (0, 0)),
            pl.BlockSpec((N_TOK, 1), lambda g: (0, 0)),
            pl.BlockSpec((N_TOK, WREP), lambda g: (0, 0)),
            pl.BlockSpec((N_TOK, WREP), lambda g: (0, 0)),
            pl.BlockSpec((1, 64), lambda g: (0, 0)),
            pl.BlockSpec((1, 1), lambda g: (0, 0)),
        ],
        out_shape=[
            jax.ShapeDtypeStruct((N_TOK, 1), jnp.int32),
            jax.ShapeDtypeStruct((N_TOK, 1), jnp.int32),
            jax.ShapeDtypeStruct((N_TOK, WREP), jnp.float32),
            jax.ShapeDtypeStruct((N_TOK, WREP), jnp.float32),
            jax.ShapeDtypeStruct((1, 64), jnp.int32),
            jax.ShapeDtypeStruct((1, 1), jnp.float32),
        ],
    )(x, Wr, br)


_SC_MESH = plsc.VectorSubcoreMesh(core_axis_name="c", subcore_axis_name="s")


@functools.partial(
    pl.kernel,
    mesh=_SC_MESH,
    out_type=[
        jax.ShapeDtypeStruct((P, D_IN), jnp.float32),
        jax.ShapeDtypeStruct((P, WREP), jnp.float32),
    ],
    scratch_types=[
        pltpu.VMEM((4, TPW // 4), jnp.int32),
        pltpu.VMEM((4, TPW // 4), jnp.int32),
        pltpu.VMEM((TPW // 4, D_IN), jnp.float32),
        pltpu.VMEM((TPW // 4, D_IN), jnp.float32),
        pltpu.VMEM((TPW, WREP), jnp.float32),
        pltpu.VMEM((TPW, WREP), jnp.float32),
        pltpu.SemaphoreType.DMA,
        pltpu.SemaphoreType.DMA,
        pltpu.SemaphoreType.DMA,
        pltpu.SemaphoreType.DMA,
        pltpu.SemaphoreType.DMA,
        pltpu.SemaphoreType.DMA,
    ],
)
def _sc_scatter(x_hbm, idx_hbm, w0_hbm, w1_hbm, xs_hbm, ws_hbm,
                idx0_v, idx1_v, r0, r1, w0f, w1f,
                rl0, rl1, sc0, sc1, sw0, sw1):
    wid = lax.axis_index("s") * 2 + lax.axis_index("c")
    base = wid * TPW
    cw = TPW // 4
    rbufs, rls, scs, sws = (r0, r1), (rl0, rl1), (sc0, sc1), (sw0, sw1)
    pltpu.sync_copy(idx_hbm.at[wid, 0], idx0_v)
    pltpu.sync_copy(idx_hbm.at[wid, 1], idx1_v)
    pltpu.sync_copy(w0_hbm.at[pl.ds(base, TPW)], w0f)
    pltpu.sync_copy(w1_hbm.at[pl.ds(base, TPW)], w1f)
    pltpu.async_copy(x_hbm.at[pl.ds(base, cw)], r0, rl0)
    for ch in range(4):
        p = ch & 1
        if ch + 1 < 4:
            q = (ch + 1) & 1
            if ch + 1 >= 2:
                pltpu.make_async_copy(
                    rbufs[q], xs_hbm.at[idx0_v.at[ch - 1]], scs[q]).wait()
                pltpu.make_async_copy(
                    rbufs[q], xs_hbm.at[idx1_v.at[ch - 1]], scs[q]).wait()
            pltpu.async_copy(
                x_hbm.at[pl.ds(base + (ch + 1) * cw, cw)], rbufs[q], rls[q])
        pltpu.make_async_copy(
            x_hbm.at[pl.ds(base + ch * cw, cw)], rbufs[p], rls[p]).wait()
        pltpu.async_copy(rbufs[p], xs_hbm.at[idx0_v.at[ch]], scs[p])
        pltpu.async_copy(rbufs[p], xs_hbm.at[idx1_v.at[ch]], scs[p])
        pltpu.async_copy(
            w0f.at[pl.ds(ch * cw, cw)], ws_hbm.at[idx0_v.at[ch]], sws[p])
        pltpu.async_copy(
            w1f.at[pl.ds(ch * cw, cw)], ws_hbm.at[idx1_v.at[ch]], sws[p])
    for ch in (2, 3):
        p = ch & 1
        pltpu.make_async_copy(
            rbufs[p], xs_hbm.at[idx0_v.at[ch]], scs[p]).wait()
        pltpu.make_async_copy(
            rbufs[p], xs_hbm.at[idx1_v.at[ch]], scs[p]).wait()
    for ch in range(4):
        p = ch & 1
        pltpu.make_async_copy(
            w0f.at[pl.ds(ch * cw, cw)], ws_hbm.at[idx0_v.at[ch]], sws[p]).wait()
        pltpu.make_async_copy(
            w1f.at[pl.ds(ch * cw, cw)], ws_hbm.at[idx1_v.at[ch]], sws[p]).wait()


def _ffn_kernel(eob_ref, xs_ref, ws_ref, w1_ref, b1_ref, w2_ref, b2_ref,
                ys_ref):
    xv = xs_ref[...]
    h = jnp.dot(xv, w1_ref[0], preferred_element_type=jnp.float32)
    h = h + b1_ref[0]
    h = 0.5 * h * (1.0 + jax.lax.erf(h * _SQRT_HALF))
    y = jnp.dot(h, w2_ref[0], preferred_element_type=jnp.float32)
    y = y + b2_ref[0]
    ys_ref[...] = y * ws_ref[:, 0:1]


def _ffn(xs, ws, eob, W1, b1, W2, b2):
    grid_spec = pltpu.PrefetchScalarGridSpec(
        num_scalar_prefetch=1,
        grid=(G,),
        in_specs=[
            pl.BlockSpec((BLK_S, D_IN), lambda g, eob: (g, 0)),
            pl.BlockSpec((BLK_S, WREP), lambda g, eob: (g, 0)),
            pl.BlockSpec((1, D_IN, HIDDEN), lambda g, eob: (eob[g], 0, 0)),
            pl.BlockSpec((1, 1, HIDDEN), lambda g, eob: (eob[g], 0, 0)),
            pl.BlockSpec((1, HIDDEN, D_IN), lambda g, eob: (eob[g], 0, 0)),
            pl.BlockSpec((1, 1, D_IN), lambda g, eob: (eob[g], 0, 0)),
        ],
        out_specs=pl.BlockSpec((BLK_S, D_IN), lambda g, eob: (g, 0)),
    )
    return pl.pallas_call(
        _ffn_kernel,
        grid_spec=grid_spec,
        out_shape=jax.ShapeDtypeStruct((P, D_IN), jnp.float32),
    )(eob, xs, ws, W1, b1.reshape(E, 1, HIDDEN), W2, b2.reshape(E, 1, D_IN))


_NCH = 4
_CW = TPW // _NCH


@functools.partial(
    pl.kernel,
    mesh=_SC_MESH,
    out_type=jax.ShapeDtypeStruct((N_TOK, D_IN), jnp.float32),
    scratch_types=[
        pltpu.VMEM((_NCH, _CW), jnp.int32),
        pltpu.VMEM((_NCH, _CW), jnp.int32),
        pltpu.VMEM((_CW, D_IN), jnp.float32),
        pltpu.VMEM((_CW, D_IN), jnp.float32),
        pltpu.VMEM((_CW, D_IN), jnp.float32),
        pltpu.VMEM((_CW, D_IN), jnp.float32),
        pltpu.SemaphoreType.DMA,
        pltpu.SemaphoreType.DMA,
        pltpu.SemaphoreType.DMA,
        pltpu.SemaphoreType.DMA,
        pltpu.SemaphoreType.DMA,
        pltpu.SemaphoreType.DMA,
    ],
)
def _sc_combine(ys_hbm, idx_hbm, out_hbm, idx0_v, idx1_v,
                a0, a1, b0, b1, ga0, ga1, gb0, gb1, st0, st1):
    wid = lax.axis_index("s") * 2 + lax.axis_index("c")
    base = wid * TPW
    abufs, bbufs = (a0, a1), (b0, b1)
    gas, gbs, sts = (ga0, ga1), (gb0, gb1), (st0, st1)
    pltpu.sync_copy(idx_hbm.at[wid, 0], idx0_v)
    pltpu.sync_copy(idx_hbm.at[wid, 1], idx1_v)
    pltpu.async_copy(ys_hbm.at[idx0_v.at[0]], a0, ga0)
    pltpu.async_copy(ys_hbm.at[idx1_v.at[0]], b0, gb0)
    for ch in range(_NCH):
        p = ch & 1
        if ch + 1 < _NCH:
            q = (ch + 1) & 1
            if ch + 1 >= 2:
                pltpu.make_async_copy(
                    abufs[q],
                    out_hbm.at[pl.ds(base + (ch - 1) * _CW, _CW)],
                    sts[q]).wait()
            pltpu.async_copy(ys_hbm.at[idx0_v.at[ch + 1]], abufs[q], gas[q])
            pltpu.async_copy(ys_hbm.at[idx1_v.at[ch + 1]], bbufs[q], gbs[q])
        pltpu.make_async_copy(ys_hbm.at[idx0_v.at[ch]], abufs[p], gas[p]).wait()
        pltpu.make_async_copy(ys_hbm.at[idx1_v.at[ch]], bbufs[p], gbs[p]).wait()
        ga, gb = abufs[p], bbufs[p]

        def tok_body(t, _, ga=ga, gb=gb):
            for k in range(D_IN // 16):
                ga[t, pl.ds(k * 16, 16)] = (
                    ga[t, pl.ds(k * 16, 16)] + gb[t, pl.ds(k * 16, 16)])
            return 0

        lax.fori_loop(0, _CW, tok_body, 0)
        pltpu.async_copy(
            abufs[p], out_hbm.at[pl.ds(base + ch * _CW, _CW)], sts[p])
    for ch in (_NCH - 2, _NCH - 1):
        p = ch & 1
        pltpu.make_async_copy(
            abufs[p], out_hbm.at[pl.ds(base + ch * _CW, _CW)], sts[p]).wait()


@jax.jit
def kernel(x, Wr, br, W1, b1, W2, b2):
    pos0, pos1, w0, w1, eob, aux = _router(x, Wr, br)
    idx_hbm = jnp.stack(
        [pos0.reshape(NW, 4, TPW // 4), pos1.reshape(NW, 4, TPW // 4)],
        axis=1)
    xs, ws = _sc_scatter(x, idx_hbm, w0, w1)
    ys = _ffn(xs, ws, eob.reshape(64)[:G], W1, b1, W2, b2)
    out = _sc_combine(ys, idx_hbm)
    return out, aux.reshape(())


# speedup vs baseline: 1.2631x; 1.0020x over previous
import functools

import jax
import jax.numpy as jnp
import numpy as np
from jax import lax
from jax.experimental import pallas as pl
from jax.experimental.pallas import tpu as pltpu
from jax.experimental.pallas import tpu_sc as plsc

D_IN = 768
E = 8
HIDDEN = 256
N_TOK = 4096
EPS = 0.1
_SQRT_HALF = 0.7071067811865476

BLK_S = 512
G = N_TOK * 2 // BLK_S + E
P = G * BLK_S
CHUNK = 512
NW = 32
TPW = N_TOK // NW
WREP = 128


def _router_kernel(x_ref, wr_ref, br_ref, pos0_ref, pos1_ref, w0_ref,
                   w1_ref, eob_ref, aux_ref):
    x = x_ref[...]
    logits = jnp.dot(x, wr_ref[...], preferred_element_type=jnp.float32)
    logits = logits + br_ref[...]
    m = jnp.max(logits, axis=-1, keepdims=True)
    ex = jnp.exp(logits - m)
    probs = ex / jnp.sum(ex, axis=-1, keepdims=True)
    probs = (1.0 - EPS) * probs + EPS / E

    iota = jax.lax.broadcasted_iota(jnp.int32, probs.shape, 1)
    m1 = jnp.max(probs, axis=-1, keepdims=True)
    e1 = jnp.min(jnp.where(probs == m1, iota, E), axis=-1, keepdims=True)
    probs2 = jnp.where(iota == e1, -jnp.inf, probs)
    m2 = jnp.max(probs2, axis=-1, keepdims=True)
    e2 = jnp.min(jnp.where(probs2 == m2, iota, E), axis=-1, keepdims=True)

    load = jnp.sum(probs, axis=0, keepdims=True) / N_TOK
    aux = jnp.sum(load * jnp.log(load * E + 1e-9)) / np.log(E + 1e-9)
    aux_ref[...] = jnp.reshape(aux, (1, 1))

    oh1 = (iota == e1).astype(jnp.float32)
    oh2 = (iota == e2).astype(jnp.float32)
    C = oh1 + oh2

    r = jax.lax.broadcasted_iota(jnp.int32, (CHUNK, CHUNK), 0)
    c = jax.lax.broadcasted_iota(jnp.int32, (CHUNK, CHUNK), 1)
    T = (r > c).astype(jnp.float32)
    run = jnp.zeros((1, E), jnp.float32)
    chunks = []
    for b in range(N_TOK // CHUNK):
        Cb = C[b * CHUNK:(b + 1) * CHUNK]
        chunks.append(jnp.dot(T, Cb, preferred_element_type=jnp.float32) + run)
        run = run + jnp.sum(Cb, axis=0, keepdims=True)
    Pm = jnp.concatenate(chunks, axis=0)

    padded = jnp.ceil(run / BLK_S) * BLK_S
    ri = jax.lax.broadcasted_iota(jnp.int32, (E, E), 0)
    ci = jax.lax.broadcasted_iota(jnp.int32, (E, E), 1)
    TU = (ri < ci).astype(jnp.float32)
    ps = jnp.dot(padded, TU, preferred_element_type=jnp.float32)

    base = ps + Pm
    pos0 = jnp.sum(oh1 * base, axis=1, keepdims=True)
    pos1 = jnp.sum(oh2 * base, axis=1, keepdims=True)
    pos0_ref[...] = pos0.astype(jnp.int32)
    pos1_ref[...] = pos1.astype(jnp.int32)
    w0_ref[...] = jnp.broadcast_to(m1, (N_TOK, WREP))
    w1_ref[...] = jnp.broadcast_to(m2, (N_TOK, WREP))

    pad_end = ps + padded
    gi = jax.lax.broadcasted_iota(
        jnp.int32, (1, 64), 1).astype(jnp.float32) * BLK_S
    acc = jnp.zeros((1, 64), jnp.float32)
    for e in range(E):
        acc = acc + (gi >= pad_end[0:1, e:e + 1]).astype(jnp.float32)
    eob_ref[...] = jnp.minimum(acc, E - 1).astype(jnp.int32)


def _router(x, Wr, br):
    return pl.pallas_call(
        _router_kernel,
        grid=(1,),
        in_specs=[
            pl.BlockSpec((N_TOK, D_IN), lambda g: (0, 0)),
            pl.BlockSpec((D_IN, E), lambda g: (0, 0)),
            pl.BlockSpec((E,), lambda g: (0,)),
        ],
        out_specs=[
            pl.BlockSpec((N_TOK, 1), lambda g: (0, 0)),
            pl.BlockSpec((N_TOK, 1), lambda g: (0, 0)),
            pl.BlockSpec((N_TOK, WREP), lambda g: (0, 0)),
            pl.BlockSpec((N_TOK, WREP), lambda g: (0, 0)),
            pl.BlockSpec((1, 64), lambda g: (0, 0)),
            pl.BlockSpec((1, 1), lambda g: (0, 0)),
        ],
        out_shape=[
            jax.ShapeDtypeStruct((N_TOK, 1), jnp.int32),
            jax.ShapeDtypeStruct((N_TOK, 1), jnp.int32),
            jax.ShapeDtypeStruct((N_TOK, WREP), jnp.float32),
            jax.ShapeDtypeStruct((N_TOK, WREP), jnp.float32),
            jax.ShapeDtypeStruct((1, 64), jnp.int32),
            jax.ShapeDtypeStruct((1, 1), jnp.float32),
        ],
    )(x, Wr, br)


_SC_MESH = plsc.VectorSubcoreMesh(core_axis_name="c", subcore_axis_name="s")


@functools.partial(
    pl.kernel,
    mesh=_SC_MESH,
    out_type=[
        jax.ShapeDtypeStruct((P, D_IN), jnp.float32),
        jax.ShapeDtypeStruct((P, WREP), jnp.float32),
    ],
    scratch_types=[
        pltpu.VMEM((4, TPW // 4), jnp.int32),
        pltpu.VMEM((4, TPW // 4), jnp.int32),
        pltpu.VMEM((TPW // 4, D_IN), jnp.float32),
        pltpu.VMEM((TPW // 4, D_IN), jnp.float32),
        pltpu.VMEM((TPW, WREP), jnp.float32),
        pltpu.VMEM((TPW, WREP), jnp.float32),
        pltpu.SemaphoreType.DMA,
        pltpu.SemaphoreType.DMA,
        pltpu.SemaphoreType.DMA,
        pltpu.SemaphoreType.DMA,
        pltpu.SemaphoreType.DMA,
        pltpu.SemaphoreType.DMA,
    ],
)
def _sc_scatter(x_hbm, idx_hbm, w0_hbm, w1_hbm, xs_hbm, ws_hbm,
                idx0_v, idx1_v, r0, r1, w0f, w1f,
                rl0, rl1, sc0, sc1, sw0, sw1):
    wid = lax.axis_index("s") * 2 + lax.axis_index("c")
    base = wid * TPW
    cw = TPW // 4
    rbufs, rls, scs, sws = (r0, r1), (rl0, rl1), (sc0, sc1), (sw0, sw1)
    pltpu.sync_copy(idx_hbm.at[wid, 0], idx0_v)
    pltpu.sync_copy(idx_hbm.at[wid, 1], idx1_v)
    pltpu.sync_copy(w0_hbm.at[pl.ds(base, TPW)], w0f)
    pltpu.sync_copy(w1_hbm.at[pl.ds(base, TPW)], w1f)
    pltpu.async_copy(x_hbm.at[pl.ds(base, cw)], r0, rl0)
    for ch in range(4):
        p = ch & 1
        if ch + 1 < 4:
            q = (ch + 1) & 1
            if ch + 1 >= 2:
                pltpu.make_async_copy(
                    rbufs[q], xs_hbm.at[idx0_v.at[ch - 1]], scs[q]).wait()
                pltpu.make_async_copy(
                    rbufs[q], xs_hbm.at[idx1_v.at[ch - 1]], scs[q]).wait()
            pltpu.async_copy(
                x_hbm.at[pl.ds(base + (ch + 1) * cw, cw)], rbufs[q], rls[q])
        pltpu.make_async_copy(
            x_hbm.at[pl.ds(base + ch * cw, cw)], rbufs[p], rls[p]).wait()
        pltpu.async_copy(rbufs[p], xs_hbm.at[idx0_v.at[ch]], scs[p])
        pltpu.async_copy(rbufs[p], xs_hbm.at[idx1_v.at[ch]], scs[p])
        pltpu.async_copy(
            w0f.at[pl.ds(ch * cw, cw)], ws_hbm.at[idx0_v.at[ch]], sws[p])
        pltpu.async_copy(
            w1f.at[pl.ds(ch * cw, cw)], ws_hbm.at[idx1_v.at[ch]], sws[p])
    for ch in (2, 3):
        p = ch & 1
        pltpu.make_async_copy(
            rbufs[p], xs_hbm.at[idx0_v.at[ch]], scs[p]).wait()
        pltpu.make_async_copy(
            rbufs[p], xs_hbm.at[idx1_v.at[ch]], scs[p]).wait()
    for ch in range(4):
        p = ch & 1
        pltpu.make_async_copy(
            w0f.at[pl.ds(ch * cw, cw)], ws_hbm.at[idx0_v.at[ch]], sws[p]).wait()
        pltpu.make_async_copy(
            w1f.at[pl.ds(ch * cw, cw)], ws_hbm.at[idx1_v.at[ch]], sws[p]).wait()


def _ffn_kernel(eob_ref, xs_ref, ws_ref, w1_ref, b1_ref, w2_ref, b2_ref,
                ys_ref):
    xv = xs_ref[...].astype(jnp.bfloat16)
    h = jnp.dot(xv, w1_ref[0].astype(jnp.bfloat16),
                preferred_element_type=jnp.float32)
    h = h + b1_ref[0]
    h = 0.5 * h * (1.0 + jax.lax.erf(h * _SQRT_HALF))
    y = jnp.dot(h.astype(jnp.bfloat16), w2_ref[0].astype(jnp.bfloat16),
                preferred_element_type=jnp.float32)
    y = y + b2_ref[0]
    ys_ref[...] = y * ws_ref[:, 0:1]


def _ffn(xs, ws, eob, W1, b1, W2, b2):
    grid_spec = pltpu.PrefetchScalarGridSpec(
        num_scalar_prefetch=1,
        grid=(G,),
        in_specs=[
            pl.BlockSpec((BLK_S, D_IN), lambda g, eob: (g, 0)),
            pl.BlockSpec((BLK_S, WREP), lambda g, eob: (g, 0)),
            pl.BlockSpec((1, D_IN, HIDDEN), lambda g, eob: (eob[g], 0, 0)),
            pl.BlockSpec((1, 1, HIDDEN), lambda g, eob: (eob[g], 0, 0)),
            pl.BlockSpec((1, HIDDEN, D_IN), lambda g, eob: (eob[g], 0, 0)),
            pl.BlockSpec((1, 1, D_IN), lambda g, eob: (eob[g], 0, 0)),
        ],
        out_specs=pl.BlockSpec((BLK_S, D_IN), lambda g, eob: (g, 0)),
    )
    return pl.pallas_call(
        _ffn_kernel,
        grid_spec=grid_spec,
        out_shape=jax.ShapeDtypeStruct((P, D_IN), jnp.float32),
    )(eob, xs, ws, W1, b1.reshape(E, 1, HIDDEN), W2, b2.reshape(E, 1, D_IN))


_NCH = 4
_CW = TPW // _NCH


@functools.partial(
    pl.kernel,
    mesh=_SC_MESH,
    out_type=jax.ShapeDtypeStruct((N_TOK, D_IN), jnp.float32),
    scratch_types=[
        pltpu.VMEM((_NCH, _CW), jnp.int32),
        pltpu.VMEM((_NCH, _CW), jnp.int32),
        pltpu.VMEM((_CW, D_IN), jnp.float32),
        pltpu.VMEM((_CW, D_IN), jnp.float32),
        pltpu.VMEM((_CW, D_IN), jnp.float32),
        pltpu.VMEM((_CW, D_IN), jnp.float32),
        pltpu.SemaphoreType.DMA,
        pltpu.SemaphoreType.DMA,
        pltpu.SemaphoreType.DMA,
        pltpu.SemaphoreType.DMA,
        pltpu.SemaphoreType.DMA,
        pltpu.SemaphoreType.DMA,
    ],
)
def _sc_combine(ys_hbm, idx_hbm, out_hbm, idx0_v, idx1_v,
                a0, a1, b0, b1, ga0, ga1, gb0, gb1, st0, st1):
    wid = lax.axis_index("s") * 2 + lax.axis_index("c")
    base = wid * TPW
    abufs, bbufs = (a0, a1), (b0, b1)
    gas, gbs, sts = (ga0, ga1), (gb0, gb1), (st0, st1)
    pltpu.sync_copy(idx_hbm.at[wid, 0], idx0_v)
    pltpu.sync_copy(idx_hbm.at[wid, 1], idx1_v)
    pltpu.async_copy(ys_hbm.at[idx0_v.at[0]], a0, ga0)
    pltpu.async_copy(ys_hbm.at[idx1_v.at[0]], b0, gb0)
    for ch in range(_NCH):
        p = ch & 1
        if ch + 1 < _NCH:
            q = (ch + 1) & 1
            if ch + 1 >= 2:
                pltpu.make_async_copy(
                    abufs[q],
                    out_hbm.at[pl.ds(base + (ch - 1) * _CW, _CW)],
                    sts[q]).wait()
            pltpu.async_copy(ys_hbm.at[idx0_v.at[ch + 1]], abufs[q], gas[q])
            pltpu.async_copy(ys_hbm.at[idx1_v.at[ch + 1]], bbufs[q], gbs[q])
        pltpu.make_async_copy(ys_hbm.at[idx0_v.at[ch]], abufs[p], gas[p]).wait()
        pltpu.make_async_copy(ys_hbm.at[idx1_v.at[ch]], bbufs[p], gbs[p]).wait()
        ga, gb = abufs[p], bbufs[p]

        def tok_body(t, _, ga=ga, gb=gb):
            for k in range(D_IN // 16):
                ga[t, pl.ds(k * 16, 16)] = (
                    ga[t, pl.ds(k * 16, 16)] + gb[t, pl.ds(k * 16, 16)])
            return 0

        lax.fori_loop(0, _CW, tok_body, 0)
        pltpu.async_copy(
            abufs[p], out_hbm.at[pl.ds(base + ch * _CW, _CW)], sts[p])
    for ch in (_NCH - 2, _NCH - 1):
        p = ch & 1
        pltpu.make_async_copy(
            abufs[p], out_hbm.at[pl.ds(base + ch * _CW, _CW)], sts[p]).wait()


@jax.jit
def kernel(x, Wr, br, W1, b1, W2, b2):
    pos0, pos1, w0, w1, eob, aux = _router(x, Wr, br)
    idx_hbm = jnp.stack(
        [pos0.reshape(NW, 4, TPW // 4), pos1.reshape(NW, 4, TPW // 4)],
        axis=1)
    xs, ws = _sc_scatter(x, idx_hbm, w0, w1)
    ys = _ffn(xs, ws, eob.reshape(64)[:G], W1, b1, W2, b2)
    out = _sc_combine(ys, idx_hbm)
    return out, aux.reshape(())
